# baseline (speedup 1.0000x reference)
import functools

import jax
import jax.numpy as jnp
from jax import lax
from jax.experimental import pallas as pl
from jax.experimental.pallas import tpu as pltpu
from jax.experimental.pallas import tpu_sc as plsc

_D = 16
_G = 8
_MT = 512
_BIGC = 1e9
_BIGD = 1e30


def _make_sc_gather(n_rows, n_idx):
    info = plsc.get_sparse_core_info()
    nw = info.num_cores * info.num_subcores
    per_w = n_idx // nw
    assert per_w * nw == n_idx and per_w % 8 == 0
    mesh = plsc.VectorSubcoreMesh(core_axis_name="c", subcore_axis_name="s")

    @functools.partial(
        pl.kernel,
        out_type=jax.ShapeDtypeStruct((n_idx, _D), jnp.float32),
        mesh=mesh,
        scratch_types=[
            pltpu.VMEM((per_w,), jnp.int32),
            pltpu.VMEM((per_w, _D), jnp.float32),
            pltpu.SemaphoreType.DMA,
        ],
        compiler_params=pltpu.CompilerParams(use_tc_tiling_on_sc=False),
    )
    def gather(table_hbm, idx_hbm, out_hbm, idx_v, rows_v, sem):
        wid = lax.axis_index("s") * info.num_cores + lax.axis_index("c")
        base = wid * per_w
        pltpu.sync_copy(idx_hbm.at[pl.ds(base, per_w)], idx_v)
        pltpu.async_copy(table_hbm.at[idx_v], rows_v, sem).wait()
        pltpu.sync_copy(rows_v, out_hbm.at[pl.ds(base, per_w)])

    return gather


def _chamfer_body(pm_ref, bl_ref, el_ref, g_ref, em_ref, out_ref, *, K, Mp, P):
    b = pl.program_id(0)
    p = pl.program_id(1)
    xl = bl_ref[b, p]
    yl = el_ref[b, p]
    KH = K // 2
    xlf = jnp.maximum(xl, 1).astype(jnp.float32)
    ylf = jnp.maximum(yl, 1).astype(jnp.float32)

    g = g_ref[0]
    X = g[:, 0:1]
    Y = g[:, 1:2]
    Z = g[:, 2:3]
    w = pm_ref[p, 8] * X + pm_ref[p, 9] * Y + pm_ref[p, 10] * Z + pm_ref[p, 11]
    u = (pm_ref[p, 0] * X + pm_ref[p, 1] * Y + pm_ref[p, 2] * Z + pm_ref[p, 3]) / w
    v = (pm_ref[p, 4] * X + pm_ref[p, 5] * Y + pm_ref[p, 6] * Z + pm_ref[p, 7]) / w

    kid = lax.broadcasted_iota(jnp.int32, (K, 1), 0)
    ucol = jnp.where(kid < xl, u, _BIGC)
    vcol = jnp.where(kid < xl, v, _BIGC)

    nt = Mp // _MT
    ntiles = (yl + _MT - 1) // _MT
    npairs = (ntiles + 1) // 2

    def run(W):
        uu = ucol[:W]
        vv = vcol[:W]

        def one_tile(i):
            chunk = em_ref[0, jnp.minimum(i, nt - 1)]
            vm = lax.broadcasted_iota(jnp.int32, (1, _MT), 1) + i * _MT < yl
            ex = jnp.where(vm, chunk[0:1, :], -_BIGC)
            ey = jnp.where(vm, chunk[1:2, :], -_BIGC)
            dx = uu - ex
            dy = vv - ey
            d2 = dx * dx + dy * dy
            mx = jnp.min(d2, axis=1, keepdims=True)
            my = jnp.where(vm, jnp.min(d2, axis=0, keepdims=True), 0.0)
            return mx, jnp.sum(my)

        def stepf(j, carry):
            minx, acc = carry
            mx0, s0 = one_tile(2 * j)
            mx1, s1 = one_tile(2 * j + 1)
            minx = jnp.minimum(minx, jnp.minimum(mx0, mx1))
            acc = acc + (s0 + s1)
            return minx, acc

        minx, acc = lax.fori_loop(
            0, npairs, stepf,
            (jnp.full((W, 1), _BIGD, dtype=jnp.float32), jnp.float32(0.0)))

        rid = lax.broadcasted_iota(jnp.int32, (W, 1), 0)
        cham_x = jnp.sum(jnp.where(rid < xl, minx, 0.0)) / xlf
        res = (cham_x + acc / ylf) * jnp.float32(1.0 / P)

        @pl.when(p == 0)
        def _():
            out_ref[b, 0] = res

        @pl.when(p > 0)
        def _():
            out_ref[b, 0] += res

    KQ = K // 4
    for wi in range(4):
        lo = wi * KQ
        hi = (wi + 1) * KQ

        @pl.when((lo < xl) & (xl <= hi))
        def _(hi=hi):
            run(hi)


def _make_tc_chamfer(B, P, K, Mp, interpret=False):
    nt = Mp // _MT
    body = functools.partial(_chamfer_body, K=K, Mp=Mp, P=P)
    return pl.pallas_call(
        body,
        grid=(B, P),
        in_specs=[
            pl.BlockSpec(memory_space=pltpu.SMEM),
            pl.BlockSpec(memory_space=pltpu.SMEM),
            pl.BlockSpec(memory_space=pltpu.SMEM),
            pl.BlockSpec((1, K, _D), lambda b, p: (b * P + p, 0, 0)),
            pl.BlockSpec((1, nt, 2, _MT), lambda b, p: (b * P + p, 0, 0, 0)),
        ],
        out_specs=pl.BlockSpec(memory_space=pltpu.SMEM),
        out_shape=jax.ShapeDtypeStruct((B, 1), jnp.float32),
        interpret=interpret,
    )


@jax.jit
def kernel(y, projmatrices, edgemaps, boundary_idx, boundary_lengths, edgemaps_len):
    B, V, _ = y.shape
    P = projmatrices.shape[0]
    M = edgemaps.shape[2]
    K = boundary_idx.shape[2]
    Mp = ((M + _MT - 1) // _MT) * _MT
    if Mp == M:
        Mp = M + _MT

    table = jnp.pad(y.reshape(B * V, 3).astype(jnp.float32), ((0, 0), (0, _D - 3)))
    idx_flat = (boundary_idx.astype(jnp.int32).reshape(B, P * K)
                + (jnp.arange(B, dtype=jnp.int32) * V)[:, None]).reshape(-1)

    gathered = _make_sc_gather(B * V, B * P * K)(table, idx_flat)
    gt = gathered.reshape(B * P, K, _D)

    em_t = jnp.moveaxis(edgemaps.astype(jnp.float32), 3, 2)
    em_t = jnp.pad(em_t, ((0, 0), (0, 0), (0, 0), (0, Mp - M)))
    em4 = em_t.reshape(B * P, 2, Mp // _MT, _MT).transpose(0, 2, 1, 3)

    pm = projmatrices.astype(jnp.float32).reshape(P, 12)
    bl = boundary_lengths.astype(jnp.int32)
    el = edgemaps_len.astype(jnp.int32)

    out = _make_tc_chamfer(B, P, K, Mp)(pm, bl, el, gt, em4)
    return out.reshape(B)


# speedup vs baseline: 1.0849x; 1.0849x over previous
import functools

import jax
import jax.numpy as jnp
from jax import lax
from jax.experimental import pallas as pl
from jax.experimental.pallas import tpu as pltpu
from jax.experimental.pallas import tpu_sc as plsc

_D = 16
_G = 8
_MT = 512
_BIGC = 1e9
_BIGD = 1e30


def _make_sc_gather(n_rows, n_idx):
    info = plsc.get_sparse_core_info()
    nw = info.num_cores * info.num_subcores
    per_w = n_idx // nw
    assert per_w * nw == n_idx and per_w % 8 == 0
    mesh = plsc.VectorSubcoreMesh(core_axis_name="c", subcore_axis_name="s")

    @functools.partial(
        pl.kernel,
        out_type=jax.ShapeDtypeStruct((n_idx, _D), jnp.float32),
        mesh=mesh,
        scratch_types=[
            pltpu.VMEM((per_w,), jnp.int32),
            pltpu.VMEM((per_w, _D), jnp.float32),
            pltpu.SemaphoreType.DMA,
        ],
        compiler_params=pltpu.CompilerParams(use_tc_tiling_on_sc=False),
    )
    def gather(table_hbm, idx_hbm, out_hbm, idx_v, rows_v, sem):
        wid = lax.axis_index("s") * info.num_cores + lax.axis_index("c")
        base = wid * per_w
        pltpu.sync_copy(idx_hbm.at[pl.ds(base, per_w)], idx_v)
        pltpu.async_copy(table_hbm.at[idx_v], rows_v, sem).wait()
        pltpu.sync_copy(rows_v, out_hbm.at[pl.ds(base, per_w)])

    return gather


def _chamfer_body(pm_ref, bl_ref, el_ref, g_ref, em_ref, out_ref, *, K, Mp, P):
    b = pl.program_id(0)
    p = pl.program_id(1)
    xl = bl_ref[b, p]
    yl = el_ref[b, p]
    KH = K // 2
    xlf = jnp.maximum(xl, 1).astype(jnp.float32)
    ylf = jnp.maximum(yl, 1).astype(jnp.float32)

    g = g_ref[0]
    X = g[0:1, :]
    Y = g[1:2, :]
    Z = g[2:3, :]
    w = pm_ref[p, 8] * X + pm_ref[p, 9] * Y + pm_ref[p, 10] * Z + pm_ref[p, 11]
    u = (pm_ref[p, 0] * X + pm_ref[p, 1] * Y + pm_ref[p, 2] * Z + pm_ref[p, 3]) / w
    v = (pm_ref[p, 4] * X + pm_ref[p, 5] * Y + pm_ref[p, 6] * Z + pm_ref[p, 7]) / w

    kid = lax.broadcasted_iota(jnp.int32, (1, K), 1)
    u = jnp.where(kid < xl, u, _BIGC)
    v = jnp.where(kid < xl, v, _BIGC)
    ucol = jnp.reshape(u, (K, 1))
    vcol = jnp.reshape(v, (K, 1))

    nt = Mp // _MT
    ntiles = (yl + _MT - 1) // _MT
    npairs = (ntiles + 1) // 2

    def run(W):
        uu = ucol[:W]
        vv = vcol[:W]

        def one_tile(i):
            chunk = em_ref[0, jnp.minimum(i, nt - 1)]
            vm = lax.broadcasted_iota(jnp.int32, (1, _MT), 1) + i * _MT < yl
            ex = jnp.where(vm, chunk[0:1, :], -_BIGC)
            ey = jnp.where(vm, chunk[1:2, :], -_BIGC)
            dx = uu - ex
            dy = vv - ey
            d2 = dx * dx + dy * dy
            mx = jnp.min(d2, axis=1, keepdims=True)
            my = jnp.where(vm, jnp.min(d2, axis=0, keepdims=True), 0.0)
            return mx, jnp.sum(my)

        def stepf(j, carry):
            minx, acc = carry
            mx0, s0 = one_tile(2 * j)
            mx1, s1 = one_tile(2 * j + 1)
            minx = jnp.minimum(minx, jnp.minimum(mx0, mx1))
            acc = acc + (s0 + s1)
            return minx, acc

        minx, acc = lax.fori_loop(
            0, npairs, stepf,
            (jnp.full((W, 1), _BIGD, dtype=jnp.float32), jnp.float32(0.0)))

        rid = lax.broadcasted_iota(jnp.int32, (W, 1), 0)
        cham_x = jnp.sum(jnp.where(rid < xl, minx, 0.0)) / xlf
        res = (cham_x + acc / ylf) * jnp.float32(1.0 / P)

        @pl.when(p == 0)
        def _():
            out_ref[b, 0] = res

        @pl.when(p > 0)
        def _():
            out_ref[b, 0] += res

    KQ = K // 8
    for wi in range(8):
        lo = wi * KQ
        hi = (wi + 1) * KQ

        @pl.when((lo < xl) & (xl <= hi))
        def _(hi=hi):
            run(hi)


def _make_tc_chamfer(B, P, K, Mp, interpret=False):
    nt = Mp // _MT
    body = functools.partial(_chamfer_body, K=K, Mp=Mp, P=P)
    return pl.pallas_call(
        body,
        grid=(B, P),
        in_specs=[
            pl.BlockSpec(memory_space=pltpu.SMEM),
            pl.BlockSpec(memory_space=pltpu.SMEM),
            pl.BlockSpec(memory_space=pltpu.SMEM),
            pl.BlockSpec((1, _G, K), lambda b, p: (b * P + p, 0, 0)),
            pl.BlockSpec((1, nt, 2, _MT), lambda b, p: (b * P + p, 0, 0, 0)),
        ],
        out_specs=pl.BlockSpec(memory_space=pltpu.SMEM),
        out_shape=jax.ShapeDtypeStruct((B, 1), jnp.float32),
        interpret=interpret,
    )


@jax.jit
def kernel(y, projmatrices, edgemaps, boundary_idx, boundary_lengths, edgemaps_len):
    B, V, _ = y.shape
    P = projmatrices.shape[0]
    M = edgemaps.shape[2]
    K = boundary_idx.shape[2]
    Mp = ((M + _MT - 1) // _MT) * _MT
    if Mp == M:
        Mp = M + _MT

    table = jnp.pad(y.reshape(B * V, 3).astype(jnp.float32), ((0, 0), (0, _D - 3)))
    idx_flat = (boundary_idx.astype(jnp.int32).reshape(B, P * K)
                + (jnp.arange(B, dtype=jnp.int32) * V)[:, None]).reshape(-1)

    gathered = _make_sc_gather(B * V, B * P * K)(table, idx_flat)
    gt = jnp.swapaxes(gathered.reshape(B * P, K, _D)[:, :, :_G], 1, 2)

    em_t = jnp.moveaxis(edgemaps.astype(jnp.float32), 3, 2)
    em_t = jnp.pad(em_t, ((0, 0), (0, 0), (0, 0), (0, Mp - M)))
    em4 = em_t.reshape(B * P, 2, Mp // _MT, _MT).transpose(0, 2, 1, 3)

    pm = projmatrices.astype(jnp.float32).reshape(P, 12)
    bl = boundary_lengths.astype(jnp.int32)
    el = edgemaps_len.astype(jnp.int32)

    out = _make_tc_chamfer(B, P, K, Mp)(pm, bl, el, gt, em4)
    return out.reshape(B)


# speedup vs baseline: 1.0857x; 1.0008x over previous
import functools

import jax
import jax.numpy as jnp
from jax import lax
from jax.experimental import pallas as pl
from jax.experimental.pallas import tpu as pltpu
from jax.experimental.pallas import tpu_sc as plsc

_D = 16
_G = 8
_MT = 512
_BIGC = 1e9
_BIGD = 1e30


def _make_sc_gather(n_rows, n_idx):
    info = plsc.get_sparse_core_info()
    nw = info.num_cores * info.num_subcores
    per_w = n_idx // nw
    assert per_w * nw == n_idx and per_w % 8 == 0
    mesh = plsc.VectorSubcoreMesh(core_axis_name="c", subcore_axis_name="s")

    @functools.partial(
        pl.kernel,
        out_type=jax.ShapeDtypeStruct((n_idx, _D), jnp.float32),
        mesh=mesh,
        scratch_types=[
            pltpu.VMEM((per_w,), jnp.int32),
            pltpu.VMEM((per_w, _D), jnp.float32),
            pltpu.SemaphoreType.DMA,
            pltpu.SemaphoreType.DMA,
            pltpu.SemaphoreType.DMA,
        ],
        compiler_params=pltpu.CompilerParams(use_tc_tiling_on_sc=False),
    )
    def gather(table_hbm, idx_hbm, out_hbm, idx_v, rows_v, sem0, sem1, sem2):
        wid = lax.axis_index("s") * info.num_cores + lax.axis_index("c")
        base = wid * per_w
        H = per_w // 2
        pltpu.sync_copy(idx_hbm.at[pl.ds(base, per_w)], idx_v)
        g0 = pltpu.async_copy(table_hbm.at[idx_v.at[pl.ds(0, H)]],
                              rows_v.at[pl.ds(0, H)], sem0)
        g1 = pltpu.async_copy(table_hbm.at[idx_v.at[pl.ds(H, H)]],
                              rows_v.at[pl.ds(H, H)], sem1)
        g0.wait()
        w0 = pltpu.async_copy(rows_v.at[pl.ds(0, H)],
                              out_hbm.at[pl.ds(base, H)], sem2)
        g1.wait()
        pltpu.sync_copy(rows_v.at[pl.ds(H, H)], out_hbm.at[pl.ds(base + H, H)])
        w0.wait()

    return gather


def _chamfer_body(pm_ref, bl_ref, el_ref, g_ref, em_ref, out_ref, *, K, Mp, P):
    b = pl.program_id(0)
    p = pl.program_id(1)
    xl = bl_ref[b, p]
    yl = el_ref[b, p]
    KH = K // 2
    xlf = jnp.maximum(xl, 1).astype(jnp.float32)
    ylf = jnp.maximum(yl, 1).astype(jnp.float32)

    g = g_ref[0]
    X = g[0:1, :]
    Y = g[1:2, :]
    Z = g[2:3, :]
    w = pm_ref[p, 8] * X + pm_ref[p, 9] * Y + pm_ref[p, 10] * Z + pm_ref[p, 11]
    u = (pm_ref[p, 0] * X + pm_ref[p, 1] * Y + pm_ref[p, 2] * Z + pm_ref[p, 3]) / w
    v = (pm_ref[p, 4] * X + pm_ref[p, 5] * Y + pm_ref[p, 6] * Z + pm_ref[p, 7]) / w

    kid = lax.broadcasted_iota(jnp.int32, (1, K), 1)
    u = jnp.where(kid < xl, u, _BIGC)
    v = jnp.where(kid < xl, v, _BIGC)
    ucol = jnp.reshape(u, (K, 1))
    vcol = jnp.reshape(v, (K, 1))

    nt = Mp // _MT
    ntiles = (yl + _MT - 1) // _MT
    npairs = (ntiles + 1) // 2

    def run(W):
        uu = ucol[:W]
        vv = vcol[:W]

        def one_tile(i):
            chunk = em_ref[0, jnp.minimum(i, nt - 1)]
            vm = lax.broadcasted_iota(jnp.int32, (1, _MT), 1) + i * _MT < yl
            ex = jnp.where(vm, chunk[0:1, :], -_BIGC)
            ey = jnp.where(vm, chunk[1:2, :], -_BIGC)
            dx = uu - ex
            dy = vv - ey
            d2 = dx * dx + dy * dy
            mx = jnp.min(d2, axis=1, keepdims=True)
            my = jnp.where(vm, jnp.min(d2, axis=0, keepdims=True), 0.0)
            return mx, jnp.sum(my)

        def stepf(j, carry):
            minx, acc = carry
            mx0, s0 = one_tile(2 * j)
            mx1, s1 = one_tile(2 * j + 1)
            minx = jnp.minimum(minx, jnp.minimum(mx0, mx1))
            acc = acc + (s0 + s1)
            return minx, acc

        minx, acc = lax.fori_loop(
            0, npairs, stepf,
            (jnp.full((W, 1), _BIGD, dtype=jnp.float32), jnp.float32(0.0)))

        rid = lax.broadcasted_iota(jnp.int32, (W, 1), 0)
        cham_x = jnp.sum(jnp.where(rid < xl, minx, 0.0)) / xlf
        res = (cham_x + acc / ylf) * jnp.float32(1.0 / P)

        @pl.when(p == 0)
        def _():
            out_ref[b, 0] = res

        @pl.when(p > 0)
        def _():
            out_ref[b, 0] += res

    KQ = K // 8
    for wi in range(8):
        lo = wi * KQ
        hi = (wi + 1) * KQ

        @pl.when((lo < xl) & (xl <= hi))
        def _(hi=hi):
            run(hi)


def _make_tc_chamfer(B, P, K, Mp, interpret=False):
    nt = Mp // _MT
    body = functools.partial(_chamfer_body, K=K, Mp=Mp, P=P)
    return pl.pallas_call(
        body,
        grid=(B, P),
        in_specs=[
            pl.BlockSpec(memory_space=pltpu.SMEM),
            pl.BlockSpec(memory_space=pltpu.SMEM),
            pl.BlockSpec(memory_space=pltpu.SMEM),
            pl.BlockSpec((1, _G, K), lambda b, p: (b * P + p, 0, 0)),
            pl.BlockSpec((1, nt, 2, _MT), lambda b, p: (b * P + p, 0, 0, 0)),
        ],
        out_specs=pl.BlockSpec(memory_space=pltpu.SMEM),
        out_shape=jax.ShapeDtypeStruct((B, 1), jnp.float32),
        interpret=interpret,
    )


@jax.jit
def kernel(y, projmatrices, edgemaps, boundary_idx, boundary_lengths, edgemaps_len):
    B, V, _ = y.shape
    P = projmatrices.shape[0]
    M = edgemaps.shape[2]
    K = boundary_idx.shape[2]
    Mp = ((M + _MT - 1) // _MT) * _MT
    if Mp == M:
        Mp = M + _MT

    table = jnp.pad(y.reshape(B * V, 3).astype(jnp.float32), ((0, 0), (0, _D - 3)))
    idx_flat = (boundary_idx.astype(jnp.int32).reshape(B, P * K)
                + (jnp.arange(B, dtype=jnp.int32) * V)[:, None]).reshape(-1)

    gathered = _make_sc_gather(B * V, B * P * K)(table, idx_flat)
    gt = jnp.swapaxes(gathered.reshape(B * P, K, _D)[:, :, :_G], 1, 2)

    em_t = jnp.moveaxis(edgemaps.astype(jnp.float32), 3, 2)
    em_t = jnp.pad(em_t, ((0, 0), (0, 0), (0, 0), (0, Mp - M)))
    em4 = em_t.reshape(B * P, 2, Mp // _MT, _MT).transpose(0, 2, 1, 3)

    pm = projmatrices.astype(jnp.float32).reshape(P, 12)
    bl = boundary_lengths.astype(jnp.int32)
    el = edgemaps_len.astype(jnp.int32)

    out = _make_tc_chamfer(B, P, K, Mp)(pm, bl, el, gt, em4)
    return out.reshape(B)


# speedup vs baseline: 1.0921x; 1.0059x over previous
import functools

import jax
import jax.numpy as jnp
from jax import lax
from jax.experimental import pallas as pl
from jax.experimental.pallas import tpu as pltpu
from jax.experimental.pallas import tpu_sc as plsc

_D = 16
_G = 8
_MT = 512
_BIGC = 1e9
_BIGD = 1e30


def _make_sc_gather(n_rows, n_idx):
    info = plsc.get_sparse_core_info()
    nw = info.num_cores * info.num_subcores
    per_w = n_idx // nw
    assert per_w * nw == n_idx and per_w % 8 == 0
    mesh = plsc.VectorSubcoreMesh(core_axis_name="c", subcore_axis_name="s")

    @functools.partial(
        pl.kernel,
        out_type=jax.ShapeDtypeStruct((n_idx, _D), jnp.float32),
        mesh=mesh,
        scratch_types=[
            pltpu.VMEM((per_w,), jnp.int32),
            pltpu.VMEM((per_w, _D), jnp.float32),
            pltpu.SemaphoreType.DMA,
        ],
        compiler_params=pltpu.CompilerParams(use_tc_tiling_on_sc=False),
    )
    def gather(table_hbm, idx_hbm, out_hbm, idx_v, rows_v, sem):
        wid = lax.axis_index("s") * info.num_cores + lax.axis_index("c")
        base = wid * per_w
        pltpu.sync_copy(idx_hbm.at[pl.ds(base, per_w)], idx_v)
        pltpu.async_copy(table_hbm.at[idx_v], rows_v, sem).wait()
        pltpu.sync_copy(rows_v, out_hbm.at[pl.ds(base, per_w)])

    return gather


def _chamfer_body(pm_ref, bl_ref, el_ref, g_ref, em_ref, out_ref, *, K, Mp, P):
    b = pl.program_id(0)
    p = pl.program_id(1)
    xl = bl_ref[b, p]
    yl = el_ref[b, p]
    KH = K // 2
    xlf = jnp.maximum(xl, 1).astype(jnp.float32)
    ylf = jnp.maximum(yl, 1).astype(jnp.float32)

    g = g_ref[0]
    X = g[0:1, :]
    Y = g[1:2, :]
    Z = g[2:3, :]
    w = pm_ref[p, 8] * X + pm_ref[p, 9] * Y + pm_ref[p, 10] * Z + pm_ref[p, 11]
    u = (pm_ref[p, 0] * X + pm_ref[p, 1] * Y + pm_ref[p, 2] * Z + pm_ref[p, 3]) / w
    v = (pm_ref[p, 4] * X + pm_ref[p, 5] * Y + pm_ref[p, 6] * Z + pm_ref[p, 7]) / w

    kid = lax.broadcasted_iota(jnp.int32, (1, K), 1)
    u = jnp.where(kid < xl, u, _BIGC)
    v = jnp.where(kid < xl, v, _BIGC)
    ucol = jnp.reshape(u, (K, 1))
    vcol = jnp.reshape(v, (K, 1))

    nt = Mp // _MT
    ntiles = (yl + _MT - 1) // _MT
    npairs = (ntiles + 1) // 2

    def run(W):
        uu = ucol[:W]
        vv = vcol[:W]

        def one_tile(i):
            chunk = em_ref[0, jnp.minimum(i, nt - 1)]
            vm = lax.broadcasted_iota(jnp.int32, (1, _MT), 1) + i * _MT < yl
            ex = jnp.where(vm, chunk[0:1, :], -_BIGC)
            ey = jnp.where(vm, chunk[1:2, :], -_BIGC)
            dx = uu - ex
            dy = vv - ey
            d2 = dx * dx + dy * dy
            mx = jnp.min(d2, axis=1, keepdims=True)
            my = jnp.where(vm, jnp.min(d2, axis=0, keepdims=True), 0.0)
            return mx, jnp.sum(my)

        def stepf(j, carry):
            minx, acc = carry
            mx0, s0 = one_tile(2 * j)
            mx1, s1 = one_tile(2 * j + 1)
            minx = jnp.minimum(minx, jnp.minimum(mx0, mx1))
            acc = acc + (s0 + s1)
            return minx, acc

        minx, acc = lax.fori_loop(
            0, npairs, stepf,
            (jnp.full((W, 1), _BIGD, dtype=jnp.float32), jnp.float32(0.0)))

        rid = lax.broadcasted_iota(jnp.int32, (W, 1), 0)
        cham_x = jnp.sum(jnp.where(rid < xl, minx, 0.0)) / xlf
        res = (cham_x + acc / ylf) * jnp.float32(1.0 / P)

        @pl.when(p == 0)
        def _():
            out_ref[b, 0] = res

        @pl.when(p > 0)
        def _():
            out_ref[b, 0] += res

    KQ = K // 16
    for wi in range(16):
        lo = wi * KQ
        hi = (wi + 1) * KQ

        @pl.when((lo < xl) & (xl <= hi))
        def _(hi=hi):
            run(hi)


def _make_tc_chamfer(B, P, K, Mp, interpret=False):
    nt = Mp // _MT
    body = functools.partial(_chamfer_body, K=K, Mp=Mp, P=P)
    return pl.pallas_call(
        body,
        grid=(B, P),
        in_specs=[
            pl.BlockSpec(memory_space=pltpu.SMEM),
            pl.BlockSpec(memory_space=pltpu.SMEM),
            pl.BlockSpec(memory_space=pltpu.SMEM),
            pl.BlockSpec((1, _G, K), lambda b, p: (b * P + p, 0, 0)),
            pl.BlockSpec((1, nt, 2, _MT), lambda b, p: (b * P + p, 0, 0, 0)),
        ],
        out_specs=pl.BlockSpec(memory_space=pltpu.SMEM),
        out_shape=jax.ShapeDtypeStruct((B, 1), jnp.float32),
        interpret=interpret,
    )


@jax.jit
def kernel(y, projmatrices, edgemaps, boundary_idx, boundary_lengths, edgemaps_len):
    B, V, _ = y.shape
    P = projmatrices.shape[0]
    M = edgemaps.shape[2]
    K = boundary_idx.shape[2]
    Mp = ((M + _MT - 1) // _MT) * _MT
    if Mp == M:
        Mp = M + _MT

    table = jnp.pad(y.reshape(B * V, 3).astype(jnp.float32), ((0, 0), (0, _D - 3)))
    idx_flat = (boundary_idx.astype(jnp.int32).reshape(B, P * K)
                + (jnp.arange(B, dtype=jnp.int32) * V)[:, None]).reshape(-1)

    gathered = _make_sc_gather(B * V, B * P * K)(table, idx_flat)
    gt = jnp.swapaxes(gathered.reshape(B * P, K, _D)[:, :, :_G], 1, 2)

    em_t = jnp.moveaxis(edgemaps.astype(jnp.float32), 3, 2)
    em_t = jnp.pad(em_t, ((0, 0), (0, 0), (0, 0), (0, Mp - M)))
    em4 = em_t.reshape(B * P, 2, Mp // _MT, _MT).transpose(0, 2, 1, 3)

    pm = projmatrices.astype(jnp.float32).reshape(P, 12)
    bl = boundary_lengths.astype(jnp.int32)
    el = edgemaps_len.astype(jnp.int32)

    out = _make_tc_chamfer(B, P, K, Mp)(pm, bl, el, gt, em4)
    return out.reshape(B)


# speedup vs baseline: 1.1324x; 1.0369x over previous
import functools

import jax
import jax.numpy as jnp
from jax import lax
from jax.experimental import pallas as pl
from jax.experimental.pallas import tpu as pltpu
from jax.experimental.pallas import tpu_sc as plsc

_D = 16
_G = 8
_MT = 512
_BIGC = 1e9
_BIGD = 1e30


def _make_sc_gather(n_rows, n_idx):
    info = plsc.get_sparse_core_info()
    nw = info.num_cores * info.num_subcores
    per_w = n_idx // nw
    assert per_w * nw == n_idx and per_w % 8 == 0
    mesh = plsc.VectorSubcoreMesh(core_axis_name="c", subcore_axis_name="s")

    @functools.partial(
        pl.kernel,
        out_type=jax.ShapeDtypeStruct((n_idx, _D), jnp.float32),
        mesh=mesh,
        scratch_types=[
            pltpu.VMEM((per_w,), jnp.int32),
            pltpu.VMEM((per_w, _D), jnp.float32),
            pltpu.SemaphoreType.DMA,
        ],
        compiler_params=pltpu.CompilerParams(use_tc_tiling_on_sc=False),
    )
    def gather(table_hbm, idx_hbm, out_hbm, idx_v, rows_v, sem):
        wid = lax.axis_index("s") * info.num_cores + lax.axis_index("c")
        base = wid * per_w
        pltpu.sync_copy(idx_hbm.at[pl.ds(base, per_w)], idx_v)
        pltpu.async_copy(table_hbm.at[idx_v], rows_v, sem).wait()
        pltpu.sync_copy(rows_v, out_hbm.at[pl.ds(base, per_w)])

    return gather


def _chamfer_body(pm_ref, bl_ref, el_ref, g_ref, em_ref, out_ref, *, K, Mp, P):
    b = pl.program_id(0)
    p = pl.program_id(1)
    xl = bl_ref[b, p]
    yl = el_ref[b, p]
    KH = K // 2
    xlf = jnp.maximum(xl, 1).astype(jnp.float32)
    ylf = jnp.maximum(yl, 1).astype(jnp.float32)

    g = g_ref[0]
    X = g[0:1, :]
    Y = g[1:2, :]
    Z = g[2:3, :]
    w = pm_ref[p, 8] * X + pm_ref[p, 9] * Y + pm_ref[p, 10] * Z + pm_ref[p, 11]
    u = (pm_ref[p, 0] * X + pm_ref[p, 1] * Y + pm_ref[p, 2] * Z + pm_ref[p, 3]) / w
    v = (pm_ref[p, 4] * X + pm_ref[p, 5] * Y + pm_ref[p, 6] * Z + pm_ref[p, 7]) / w

    kid = lax.broadcasted_iota(jnp.int32, (1, K), 1)
    u = jnp.where(kid < xl, u, _BIGC)
    v = jnp.where(kid < xl, v, _BIGC)
    ucol = jnp.reshape(u, (K, 1))
    vcol = jnp.reshape(v, (K, 1))

    nt = Mp // _MT
    ntiles = (yl + _MT - 1) // _MT
    npairs = ntiles // 2

    def run(W):
        uu = ucol[:W]
        vv = vcol[:W]

        def one_tile(i):
            chunk = em_ref[0, jnp.minimum(i, nt - 1)]
            vm = lax.broadcasted_iota(jnp.int32, (1, _MT), 1) + i * _MT < yl
            ex = jnp.where(vm, chunk[0:1, :], -_BIGC)
            ey = jnp.where(vm, chunk[1:2, :], -_BIGC)
            dx = uu - ex
            dy = vv - ey
            d2 = dx * dx + dy * dy
            mx = jnp.min(d2, axis=1, keepdims=True)
            my = jnp.where(vm, jnp.min(d2, axis=0, keepdims=True), 0.0)
            return mx, jnp.sum(my)

        def stepf(j, carry):
            minx, acc = carry
            mx0, s0 = one_tile(2 * j)
            mx1, s1 = one_tile(2 * j + 1)
            minx = jnp.minimum(minx, jnp.minimum(mx0, mx1))
            acc = acc + (s0 + s1)
            return minx, acc

        def step_one(i, carry):
            minx, acc = carry
            mx0, s0 = one_tile(i)
            return jnp.minimum(minx, mx0), acc + s0

        minx, acc = lax.fori_loop(
            0, npairs, stepf,
            (jnp.full((W, 1), _BIGD, dtype=jnp.float32), jnp.float32(0.0)))
        minx, acc = lax.fori_loop(2 * npairs, ntiles, step_one, (minx, acc))

        rid = lax.broadcasted_iota(jnp.int32, (W, 1), 0)
        cham_x = jnp.sum(jnp.where(rid < xl, minx, 0.0)) / xlf
        res = (cham_x + acc / ylf) * jnp.float32(1.0 / P)

        @pl.when(p == 0)
        def _():
            out_ref[b, 0] = res

        @pl.when(p > 0)
        def _():
            out_ref[b, 0] += res

    KQ = K // 16
    for wi in range(16):
        lo = wi * KQ
        hi = (wi + 1) * KQ

        @pl.when((lo < xl) & (xl <= hi))
        def _(hi=hi):
            run(hi)


def _make_tc_chamfer(B, P, K, Mp, interpret=False):
    nt = Mp // _MT
    body = functools.partial(_chamfer_body, K=K, Mp=Mp, P=P)
    return pl.pallas_call(
        body,
        grid=(B, P),
        in_specs=[
            pl.BlockSpec(memory_space=pltpu.SMEM),
            pl.BlockSpec(memory_space=pltpu.SMEM),
            pl.BlockSpec(memory_space=pltpu.SMEM),
            pl.BlockSpec((1, _G, K), lambda b, p: (b * P + p, 0, 0)),
            pl.BlockSpec((1, nt, 2, _MT), lambda b, p: (b * P + p, 0, 0, 0)),
        ],
        out_specs=pl.BlockSpec(memory_space=pltpu.SMEM),
        out_shape=jax.ShapeDtypeStruct((B, 1), jnp.float32),
        interpret=interpret,
    )


@jax.jit
def kernel(y, projmatrices, edgemaps, boundary_idx, boundary_lengths, edgemaps_len):
    B, V, _ = y.shape
    P = projmatrices.shape[0]
    M = edgemaps.shape[2]
    K = boundary_idx.shape[2]
    Mp = ((M + _MT - 1) // _MT) * _MT
    if Mp == M:
        Mp = M + _MT

    table = jnp.pad(y.reshape(B * V, 3).astype(jnp.float32), ((0, 0), (0, _D - 3)))
    idx_flat = (boundary_idx.astype(jnp.int32).reshape(B, P * K)
                + (jnp.arange(B, dtype=jnp.int32) * V)[:, None]).reshape(-1)

    gathered = _make_sc_gather(B * V, B * P * K)(table, idx_flat)
    gt = jnp.swapaxes(gathered.reshape(B * P, K, _D)[:, :, :_G], 1, 2)

    em_t = jnp.moveaxis(edgemaps.astype(jnp.float32), 3, 2)
    em_t = jnp.pad(em_t, ((0, 0), (0, 0), (0, 0), (0, Mp - M)))
    em4 = em_t.reshape(B * P, 2, Mp // _MT, _MT).transpose(0, 2, 1, 3)

    pm = projmatrices.astype(jnp.float32).reshape(P, 12)
    bl = boundary_lengths.astype(jnp.int32)
    el = edgemaps_len.astype(jnp.int32)

    out = _make_tc_chamfer(B, P, K, Mp)(pm, bl, el, gt, em4)
    return out.reshape(B)
